# baseline (speedup 1.0000x reference)
import functools

import jax
import jax.numpy as jnp
from jax import lax
from jax.experimental import pallas as pl
from jax.experimental.pallas import tpu as pltpu
from jax.experimental.pallas import tpu_sc as plsc

J = 512
M = 2
JPM = J + M
FLAT = J * JPM

NBLK = 8
CSC = 131072
TCOLS = FLAT - CSC
TBLKW = TCOLS // NBLK
NSUB = 32
PERT = CSC // NSUB
KCH = 8
NCH = 128 // KCH
VB = 512
NV = VB // 16
NVB = PERT // VB

_HI = jax.lax.Precision.HIGHEST
_DEF = jax.lax.Precision.DEFAULT


def _prelim_kernel(g_ref, hr_ref, hc_ref, lr_ref, wpn_ref,
                   wjj_ref, bjj_ref, wjm_ref, bjm_ref,
                   wpool_ref, bpool_ref, wself_ref, wneigh_ref, bsage_ref,
                   wlj_ref, blj_ref,
                   term_ref, base_ref, penl_ref, penr_ref, mf_ref):
    g = g_ref[...]
    hr = hr_ref[...]
    hc = hc_ref[...]
    lr = lr_ref[...]
    wpn = wpn_ref[...]

    ii = jax.lax.broadcasted_iota(jnp.int32, (J, J), 0)
    jj = jax.lax.broadcasted_iota(jnp.int32, (J, J), 1)
    gt = (hc > hr).astype(jnp.float32)
    tie = ((hc == hr) & (ii < jj)).astype(jnp.float32)
    rank_row = jnp.sum(gt + tie, axis=0, keepdims=True)

    onehot = (rank_row == ii.astype(jnp.float32)).astype(jnp.float32)
    sorted_h = jnp.sum(onehot * hr, axis=1, keepdims=True)
    sorted_l = jnp.sum(onehot * lr, axis=1, keepdims=True)

    a = g[:, :J]
    b = g[:, J:]
    onesc = jnp.ones((J, 1), jnp.float32)

    def aggT(adj, v):
        return jax.lax.dot_general(adj, v, (((0,), (0,)), ((), ())),
                                   precision=_HI)

    ds_jj = jax.lax.rsqrt(jnp.clip(jnp.sum(a, axis=1, keepdims=True), 1.0))
    dd_jj = jax.lax.rsqrt(jnp.clip(aggT(a, onesc), 1.0))
    agg_h = aggT(a, sorted_h * ds_jj) * dd_jj
    agg_l = aggT(a, sorted_l * ds_jj) * dd_jj
    agg_c = aggT(a, ds_jj) * dd_jj
    wjj = wjj_ref[...]
    wc = (wpn[0, 0] * wjj[2:3, :] + wpn[0, 1] * wjj[3:4, :]
          + wpn[0, 2] * wjj[4:5, :])
    job_conv = (agg_h * wjj[0:1, :] + agg_l * wjj[1:2, :] + agg_c * wc
                + bjj_ref[...])

    ds_jm = jax.lax.rsqrt(jnp.clip(jnp.sum(b, axis=1, keepdims=True), 1.0))
    dd_jm = jax.lax.rsqrt(jnp.clip(aggT(b, onesc), 1.0))
    aggm_h = aggT(b, sorted_h * ds_jm) * dd_jm
    aggm_l = aggT(b, sorted_l * ds_jm) * dd_jm
    aggm_c = aggT(b, ds_jm) * dd_jm
    wjm = wjm_ref[...]
    wcm = (wpn[0, 0] * wjm[2:3, :] + wpn[0, 1] * wjm[3:4, :]
           + wpn[0, 2] * wjm[4:5, :])
    mc = (aggm_h * wjm[0:1, :] + aggm_l * wjm[1:2, :]
          + aggm_c * wcm + bjm_ref[...])

    o64 = jax.lax.broadcasted_iota(jnp.int32, (64, 128), 0)
    n64 = jax.lax.broadcasted_iota(jnp.int32, (64, 128), 1)
    pmat = (o64 == n64 % 64).astype(jnp.float32)
    m2 = jax.lax.broadcasted_iota(jnp.int32, (M, 128), 0)
    n2 = jax.lax.broadcasted_iota(jnp.int32, (M, 128), 1)
    qmat = (m2 == n2 // 64).astype(jnp.float32)
    mcp = jax.lax.dot_general(mc, pmat, (((1,), (0,)), ((), ())),
                              precision=_HI)
    mf_ref[...] = jnp.sum(qmat * mcp, axis=0, keepdims=True)

    h_pool = jax.nn.relu(jnp.sum(wpool_ref[...], axis=0, keepdims=True)
                         + bpool_ref[...])
    term_ref[...] = (jnp.sum(wself_ref[...], axis=0, keepdims=True)
                     + jax.lax.dot_general(h_pool, wneigh_ref[...],
                                           (((1,), (0,)), ((), ())),
                                           precision=_HI)
                     + bsage_ref[...])

    base_ref[...] = jax.lax.dot_general(job_conv, wlj_ref[...],
                                        (((1,), (0,)), ((), ())),
                                        precision=_HI) + blj_ref[...]

    row = jnp.sum(g, axis=1, keepdims=True)
    col_row = jax.lax.dot_general(jnp.ones((1, J), jnp.float32), g,
                                  (((1,), (0,)), ((), ())),
                                  precision=_HI)
    rowT_row = jax.lax.dot_general(jnp.ones((1, JPM), jnp.float32), g,
                                   (((1,), (1,)), ((), ())),
                                   precision=_HI)
    left = (jnp.ones((J, J), jnp.float32) - row - rowT_row
            - col_row[:, :J] - aggT(a, onesc))
    leftb = jnp.where(left == 1.0, 1.0, 0.0)
    leftb = jnp.where(jj > ii, leftb, 0.0)
    penl_ref[...] = (1.0 - leftb) * 100000.0
    penr_ref[...] = jnp.broadcast_to(row, (J, M)) * 100000.0


def _tc_stream_kernel(mf_ref, wlma_ref, wlmb_ref, blm_ref, vmach_ref):
    mf = mf_ref[...]
    part = (jax.lax.dot_general(mf[:, :64], wlma_ref[...],
                                (((1,), (0,)), ((), ())), precision=_DEF)
            + jax.lax.dot_general(mf[:, 64:], wlmb_ref[...],
                                  (((1,), (0,)), ((), ())), precision=_DEF))
    vmach_ref[0] = part + blm_ref[0]


@functools.cache
def _get_sc_matvec():
    mesh = plsc.VectorSubcoreMesh(core_axis_name="c", subcore_axis_name="s")

    @functools.partial(
        pl.kernel, mesh=mesh,
        out_type=jax.ShapeDtypeStruct((CSC,), jnp.float32),
        scratch_types=[
            pltpu.VMEM((144,), jnp.float32),
            pltpu.VMEM((KCH, PERT), jnp.float32),
            pltpu.VMEM((KCH, PERT), jnp.float32),
            pltpu.VMEM((PERT,), jnp.float32),
            pltpu.SemaphoreType.DMA,
            pltpu.SemaphoreType.DMA,
        ],
    )
    def _sc_matvec(mf_hbm, wlm_hbm, blm_hbm, out_hbm,
                   mf_v, wa_v, wb_v, acc_v, sema, semb):
        wid = lax.axis_index("s") * 2 + lax.axis_index("c")
        col0 = TCOLS + wid * PERT
        pltpu.sync_copy(mf_hbm, mf_v.at[pl.ds(0, 128)])
        pltpu.sync_copy(blm_hbm.at[pl.ds(col0, PERT)], acc_v)

        def compute_chunk(ch, w_v):
            mfg = mf_v[pl.ds(ch * KCH, 16)]

            def vstep(vb, _):
                accs = [acc_v[pl.ds(vb * VB + v * 16, 16)] for v in range(NV)]
                for l in range(KCH):
                    s = mfg[l]
                    for v in range(NV):
                        accs[v] = accs[v] + s * w_v[l, pl.ds(vb * VB + v * 16,
                                                             16)]
                for v in range(NV):
                    acc_v[pl.ds(vb * VB + v * 16, 16)] = accs[v]
                return 0

            lax.fori_loop(0, NVB, vstep, 0)

        def wcopy(ch, buf, sem):
            return pltpu.async_copy(
                wlm_hbm.at[pl.ds(ch * KCH, KCH), pl.ds(col0, PERT)], buf, sem)

        cp0 = wcopy(0, wa_v, sema)
        cp0.wait()

        def pair(i, _):
            cpb = wcopy(2 * i + 1, wb_v, semb)
            compute_chunk(2 * i, wa_v)
            cpb.wait()

            @pl.when(i < NCH // 2 - 1)
            def _():
                cpa = wcopy(2 * i + 2, wa_v, sema)
                compute_chunk(2 * i + 1, wb_v)
                cpa.wait()

            @pl.when(i >= NCH // 2 - 1)
            def _():
                compute_chunk(2 * i + 1, wb_v)

            return 0

        lax.fori_loop(0, NCH // 2, pair, 0)
        pltpu.sync_copy(acc_v, out_hbm.at[pl.ds(wid * PERT, PERT)])

    return _sc_matvec


def _finish_kernel(vm_ref, base_ref, penl_ref, penr_ref, val_ref, poss_ref):
    v = vm_ref[...] + base_ref[...]
    val_ref[...] = v
    tl = v[:, :J] - penl_ref[...]
    tr = v[:, J:] - penr_ref[...]
    m = jnp.maximum(jnp.max(tl), jnp.max(tr))
    el = jnp.exp(tl - m)
    er = jnp.exp(tr - m)
    s = jnp.sum(el) + jnp.sum(er)
    poss_ref[:, :J] = el / s
    poss_ref[:, J:] = er / s


@functools.partial(jax.jit, static_argnames=())
def kernel(Graph, h, L, W, P, N, jj_src, jj_dst, jm_src, jm_dst,
           W_jj, b_jj, W_jm, b_jm, W_pool, b_pool, W_self, W_neigh, b_sage,
           W_lj, b_lj, W_lm, b_lm):
    del jj_src, jj_dst, jm_src, jm_dst
    f32 = jnp.float32
    hr = h.reshape(1, J).astype(f32)
    hc = h.reshape(J, 1).astype(f32)
    lr = L.reshape(1, J).astype(f32)
    wpn = jnp.stack([W, P, N]).reshape(1, 3).astype(f32)

    const = lambda shape: pl.BlockSpec(shape, lambda j: tuple(0 for _ in shape))
    term, base, penl, penr, mf = pl.pallas_call(
        _prelim_kernel,
        out_shape=(
            jax.ShapeDtypeStruct((1, 1), f32),
            jax.ShapeDtypeStruct((J, JPM), f32),
            jax.ShapeDtypeStruct((J, J), f32),
            jax.ShapeDtypeStruct((J, M), f32),
            jax.ShapeDtypeStruct((1, 128), f32),
        ),
    )(Graph, hr, hc, lr, wpn,
      W_jj, b_jj.reshape(1, -1), W_jm, b_jm.reshape(1, -1),
      W_pool, b_pool.reshape(1, -1), W_self, W_neigh, b_sage.reshape(1, 1),
      W_lj, b_lj.reshape(1, -1))

    blm_tc = lax.slice(b_lm, (0,), (TCOLS,)).reshape(NBLK, 1, TBLKW)

    vmach_tc = pl.pallas_call(
        _tc_stream_kernel,
        grid=(NBLK,),
        in_specs=[
            const((1, 128)),
            pl.BlockSpec((64, TBLKW), lambda j: (0, j)),
            pl.BlockSpec((64, TBLKW), lambda j: (1, j)),
            pl.BlockSpec((1, 1, TBLKW), lambda j: (j, 0, 0)),
        ],
        out_specs=pl.BlockSpec((1, 1, TBLKW), lambda j: (j, 0, 0)),
        out_shape=jax.ShapeDtypeStruct((NBLK, 1, TBLKW), f32),
    )(mf, W_lm, W_lm, blm_tc)

    vmach_sc = _get_sc_matvec()(mf.reshape(128), W_lm, b_lm)

    vmach2d = jnp.concatenate(
        [vmach_tc.reshape(TCOLS), vmach_sc], axis=0).reshape(J, JPM)
    value, poss = pl.pallas_call(
        _finish_kernel,
        out_shape=(jax.ShapeDtypeStruct((J, JPM), f32),
                   jax.ShapeDtypeStruct((J, JPM), f32)),
    )(vmach2d, base, penl, penr)

    return (term, value, poss)


# speedup vs baseline: 1.5960x; 1.5960x over previous
import functools

import jax
import jax.numpy as jnp
from jax import lax
from jax.experimental import pallas as pl
from jax.experimental.pallas import tpu as pltpu
from jax.experimental.pallas import tpu_sc as plsc

J = 512
M = 2
JPM = J + M
FLAT = J * JPM
NBLK = 8
BLKW = FLAT // NBLK

_HI = jax.lax.Precision.HIGHEST
_DEF = jax.lax.Precision.DEFAULT


def _stream_kernel(g_ref, hr_ref, hc_ref, lr_ref, wpn_ref,
                   wjj_ref, bjj_ref, wjm_ref, bjm_ref,
                   wlj_ref, blj_ref, wlma_ref, wlmb_ref, blm_ref,
                   base_ref, penl_ref, penr_ref, vmach_ref,
                   mf_ref):
    @pl.when(pl.program_id(0) == 0)
    def _prelim():
        g = g_ref[...]
        hr = hr_ref[...]
        hc = hc_ref[...]
        lr = lr_ref[...]
        wpn = wpn_ref[...]

        ii = jax.lax.broadcasted_iota(jnp.int32, (J, J), 0)
        jj = jax.lax.broadcasted_iota(jnp.int32, (J, J), 1)
        gt = (hc > hr).astype(jnp.float32)
        tie = ((hc == hr) & (ii < jj)).astype(jnp.float32)
        rank_row = jnp.sum(gt + tie, axis=0, keepdims=True)

        onehot = (rank_row == ii.astype(jnp.float32)).astype(jnp.float32)
        sorted_h = jnp.sum(onehot * hr, axis=1, keepdims=True)
        sorted_l = jnp.sum(onehot * lr, axis=1, keepdims=True)

        a = g[:, :J]
        b = g[:, J:]
        onesc = jnp.ones((J, 1), jnp.float32)

        def aggT(adj, v):
            return jax.lax.dot_general(adj, v, (((0,), (0,)), ((), ())),
                                       precision=_HI)

        ds_jj = jax.lax.rsqrt(jnp.clip(jnp.sum(a, axis=1, keepdims=True), 1.0))
        dd_jj = jax.lax.rsqrt(jnp.clip(aggT(a, onesc), 1.0))
        agg_h = aggT(a, sorted_h * ds_jj) * dd_jj
        agg_l = aggT(a, sorted_l * ds_jj) * dd_jj
        agg_c = aggT(a, ds_jj) * dd_jj
        wjj = wjj_ref[...]
        wc = (wpn[0, 0] * wjj[2:3, :] + wpn[0, 1] * wjj[3:4, :]
              + wpn[0, 2] * wjj[4:5, :])
        job_conv = (agg_h * wjj[0:1, :] + agg_l * wjj[1:2, :] + agg_c * wc
                    + bjj_ref[...])

        ds_jm = jax.lax.rsqrt(jnp.clip(jnp.sum(b, axis=1, keepdims=True), 1.0))
        dd_jm = jax.lax.rsqrt(jnp.clip(aggT(b, onesc), 1.0))
        aggm_h = aggT(b, sorted_h * ds_jm) * dd_jm
        aggm_l = aggT(b, sorted_l * ds_jm) * dd_jm
        aggm_c = aggT(b, ds_jm) * dd_jm
        wjm = wjm_ref[...]
        wcm = (wpn[0, 0] * wjm[2:3, :] + wpn[0, 1] * wjm[3:4, :]
               + wpn[0, 2] * wjm[4:5, :])
        mc = (aggm_h * wjm[0:1, :] + aggm_l * wjm[1:2, :]
              + aggm_c * wcm + bjm_ref[...])

        o64 = jax.lax.broadcasted_iota(jnp.int32, (64, 128), 0)
        n64 = jax.lax.broadcasted_iota(jnp.int32, (64, 128), 1)
        pmat = (o64 == n64 % 64).astype(jnp.float32)
        m2 = jax.lax.broadcasted_iota(jnp.int32, (M, 128), 0)
        n2 = jax.lax.broadcasted_iota(jnp.int32, (M, 128), 1)
        qmat = (m2 == n2 // 64).astype(jnp.float32)
        mcp = jax.lax.dot_general(mc, pmat, (((1,), (0,)), ((), ())),
                                  precision=_HI)
        mf_ref[...] = jnp.sum(qmat * mcp, axis=0, keepdims=True)

        base_ref[...] = jax.lax.dot_general(job_conv, wlj_ref[...],
                                            (((1,), (0,)), ((), ())),
                                            precision=_HI) + blj_ref[...]

        row = jnp.sum(g, axis=1, keepdims=True)
        col_row = jax.lax.dot_general(jnp.ones((1, J), jnp.float32), g,
                                      (((1,), (0,)), ((), ())),
                                      precision=_HI)
        rowT_row = jax.lax.dot_general(jnp.ones((1, JPM), jnp.float32), g,
                                       (((1,), (1,)), ((), ())),
                                       precision=_HI)
        left = (jnp.ones((J, J), jnp.float32) - row - rowT_row
                - col_row[:, :J] - aggT(a, onesc))
        leftb = jnp.where(left == 1.0, 1.0, 0.0)
        leftb = jnp.where(jj > ii, leftb, 0.0)
        penl_ref[...] = (1.0 - leftb) * 100000.0
        penr_ref[...] = jnp.broadcast_to(row, (J, M)) * 100000.0

    mf = mf_ref[...]
    part = (jax.lax.dot_general(mf[:, :64], wlma_ref[...],
                                (((1,), (0,)), ((), ())), precision=_DEF)
            + jax.lax.dot_general(mf[:, 64:], wlmb_ref[...],
                                  (((1,), (0,)), ((), ())), precision=_DEF))
    vmach_ref[0] = part + blm_ref[0]


@functools.cache
def _get_sc_terminal():
    mesh = plsc.VectorSubcoreMesh(core_axis_name="c", subcore_axis_name="s")

    @functools.partial(
        pl.kernel, mesh=mesh,
        out_type=jax.ShapeDtypeStruct((16,), jnp.float32),
        scratch_types=[pltpu.VMEM((128,), jnp.float32),
                       pltpu.VMEM((16,), jnp.float32)],
    )
    def _sc_terminal(wpad_hbm, out_hbm, w_v, o_v):
        wid = lax.axis_index("s") * 2 + lax.axis_index("c")

        @pl.when(wid == 0)
        def _():
            pltpu.sync_copy(wpad_hbm, w_v)
            row = lambda r: w_v[pl.ds(16 * r, 16)]
            hp = jnp.maximum(
                row(0) + row(1) + row(2) + row(3) + row(4) + row(5), 0.0)
            hp = jnp.maximum(hp, hp)
            z = hp * row(6) + row(7)
            t = z[0]
            for l in range(1, 16):
                t = t + z[l]
            o_v[...] = (z * 0.0 + 1.0) * t
            pltpu.sync_copy(o_v, out_hbm)

    return _sc_terminal


def _finish_kernel(vm_ref, base_ref, penl_ref, penr_ref, val_ref, poss_ref):
    v = vm_ref[...] + base_ref[...]
    val_ref[...] = v
    tl = v[:, :J] - penl_ref[...]
    tr = v[:, J:] - penr_ref[...]
    m = jnp.maximum(jnp.max(tl), jnp.max(tr))
    el = jnp.exp(tl - m)
    er = jnp.exp(tr - m)
    s = jnp.sum(el) + jnp.sum(er)
    poss_ref[:, :J] = el / s
    poss_ref[:, J:] = er / s


@functools.partial(jax.jit, static_argnames=())
def kernel(Graph, h, L, W, P, N, jj_src, jj_dst, jm_src, jm_dst,
           W_jj, b_jj, W_jm, b_jm, W_pool, b_pool, W_self, W_neigh, b_sage,
           W_lj, b_lj, W_lm, b_lm):
    del jj_src, jj_dst, jm_src, jm_dst
    f32 = jnp.float32
    hr = h.reshape(1, J).astype(f32)
    hc = h.reshape(J, 1).astype(f32)
    lr = L.reshape(1, J).astype(f32)
    wpn = jnp.stack([W, P, N]).reshape(1, 3).astype(f32)
    blm_flat = b_lm.reshape(NBLK, 1, BLKW)

    wpad = jnp.zeros((8, 16), f32)
    wpad = wpad.at[0:5, 0:5].set(W_pool.astype(f32))
    wpad = wpad.at[5, 0:5].set(b_pool.astype(f32))
    wpad = wpad.at[6, 0:5].set(W_neigh[:, 0].astype(f32))
    wpad = wpad.at[7, 0:5].set(W_self[:, 0].astype(f32))
    wpad = wpad.at[7, 15].set(b_sage[0].astype(f32))
    term = _get_sc_terminal()(wpad.reshape(128))[0].reshape(1, 1)

    const = lambda shape: pl.BlockSpec(shape, lambda j: tuple(0 for _ in shape))
    base, penl, penr, vmach = pl.pallas_call(
        _stream_kernel,
        grid=(NBLK,),
        in_specs=[
            const((J, JPM)), const((1, J)), const((J, 1)), const((1, J)),
            const((1, 3)),
            const((5, 256)), const((1, 256)), const((5, 64)), const((1, 64)),
            const((256, JPM)), const((1, JPM)),
            pl.BlockSpec((64, BLKW), lambda j: (0, j)),
            pl.BlockSpec((64, BLKW), lambda j: (1, j)),
            pl.BlockSpec((1, 1, BLKW), lambda j: (j, 0, 0)),
        ],
        out_specs=(
            const((J, JPM)), const((J, J)), const((J, M)),
            pl.BlockSpec((1, 1, BLKW), lambda j: (j, 0, 0)),
        ),
        out_shape=(
            jax.ShapeDtypeStruct((J, JPM), f32),
            jax.ShapeDtypeStruct((J, J), f32),
            jax.ShapeDtypeStruct((J, M), f32),
            jax.ShapeDtypeStruct((NBLK, 1, BLKW), f32),
        ),
        scratch_shapes=[pltpu.VMEM((1, 128), f32)],
    )(Graph, hr, hc, lr, wpn,
      W_jj, b_jj.reshape(1, -1), W_jm, b_jm.reshape(1, -1),
      W_lj, b_lj.reshape(1, -1), W_lm, W_lm, blm_flat)

    vmach2d = vmach.reshape(J, JPM)
    value, poss = pl.pallas_call(
        _finish_kernel,
        out_shape=(jax.ShapeDtypeStruct((J, JPM), f32),
                   jax.ShapeDtypeStruct((J, JPM), f32)),
    )(vmach2d, base, penl, penr)

    return (term, value, poss)


# speedup vs baseline: 1.8848x; 1.1810x over previous
import functools

import jax
import jax.numpy as jnp
from jax.experimental import pallas as pl
from jax.experimental.pallas import tpu as pltpu

J = 512
M = 2
JPM = J + M
FLAT = J * JPM
NBLK = 8
BLKW = FLAT // NBLK

_HI = jax.lax.Precision.HIGHEST
_DEF = jax.lax.Precision.DEFAULT


def _stream_kernel(g_ref, hr_ref, hc_ref, lr_ref, wpn_ref,
                   wjj_ref, bjj_ref, wjm_ref, bjm_ref,
                   wpool_ref, bpool_ref, wself_ref, wneigh_ref, bsage_ref,
                   wlj_ref, blj_ref, wlma_ref, wlmb_ref, blm_ref,
                   term_ref, base_ref, penl_ref, penr_ref, vmach_ref,
                   mf_ref):
    @pl.when(pl.program_id(0) == 0)
    def _prelim():
        g = g_ref[...]
        hr = hr_ref[...]
        hc = hc_ref[...]
        lr = lr_ref[...]
        wpn = wpn_ref[...]

        ii = jax.lax.broadcasted_iota(jnp.int32, (J, J), 0)
        jj = jax.lax.broadcasted_iota(jnp.int32, (J, J), 1)
        gt = (hc > hr).astype(jnp.float32)
        tie = ((hc == hr) & (ii < jj)).astype(jnp.float32)
        rank_row = jnp.sum(gt + tie, axis=0, keepdims=True)

        onehot = (rank_row == ii.astype(jnp.float32)).astype(jnp.float32)
        sorted_h = jnp.sum(onehot * hr, axis=1, keepdims=True)
        sorted_l = jnp.sum(onehot * lr, axis=1, keepdims=True)

        a = g[:, :J]
        b = g[:, J:]
        onesc = jnp.ones((J, 1), jnp.float32)

        def aggT(adj, v):
            return jax.lax.dot_general(adj, v, (((0,), (0,)), ((), ())),
                                       precision=_HI)

        ds_jj = jax.lax.rsqrt(jnp.clip(jnp.sum(a, axis=1, keepdims=True), 1.0))
        dd_jj = jax.lax.rsqrt(jnp.clip(aggT(a, onesc), 1.0))
        agg_h = aggT(a, sorted_h * ds_jj) * dd_jj
        agg_l = aggT(a, sorted_l * ds_jj) * dd_jj
        agg_c = aggT(a, ds_jj) * dd_jj
        wjj = wjj_ref[...]
        wc = (wpn[0, 0] * wjj[2:3, :] + wpn[0, 1] * wjj[3:4, :]
              + wpn[0, 2] * wjj[4:5, :])
        job_conv = (agg_h * wjj[0:1, :] + agg_l * wjj[1:2, :] + agg_c * wc
                    + bjj_ref[...])

        ds_jm = jax.lax.rsqrt(jnp.clip(jnp.sum(b, axis=1, keepdims=True), 1.0))
        dd_jm = jax.lax.rsqrt(jnp.clip(aggT(b, onesc), 1.0))
        aggm_h = aggT(b, sorted_h * ds_jm) * dd_jm
        aggm_l = aggT(b, sorted_l * ds_jm) * dd_jm
        aggm_c = aggT(b, ds_jm) * dd_jm
        wjm = wjm_ref[...]
        wcm = (wpn[0, 0] * wjm[2:3, :] + wpn[0, 1] * wjm[3:4, :]
               + wpn[0, 2] * wjm[4:5, :])
        mc = (aggm_h * wjm[0:1, :] + aggm_l * wjm[1:2, :]
              + aggm_c * wcm + bjm_ref[...])

        o64 = jax.lax.broadcasted_iota(jnp.int32, (64, 128), 0)
        n64 = jax.lax.broadcasted_iota(jnp.int32, (64, 128), 1)
        pmat = (o64 == n64 % 64).astype(jnp.float32)
        m2 = jax.lax.broadcasted_iota(jnp.int32, (M, 128), 0)
        n2 = jax.lax.broadcasted_iota(jnp.int32, (M, 128), 1)
        qmat = (m2 == n2 // 64).astype(jnp.float32)
        mcp = jax.lax.dot_general(mc, pmat, (((1,), (0,)), ((), ())),
                                  precision=_HI)
        mf_ref[...] = jnp.sum(qmat * mcp, axis=0, keepdims=True)

        h_pool = jax.nn.relu(jnp.sum(wpool_ref[...], axis=0, keepdims=True)
                             + bpool_ref[...])
        term_ref[...] = (jnp.sum(wself_ref[...], axis=0, keepdims=True)
                         + jax.lax.dot_general(h_pool, wneigh_ref[...],
                                               (((1,), (0,)), ((), ())),
                                               precision=_HI)
                         + bsage_ref[...])

        base_ref[...] = jax.lax.dot_general(job_conv, wlj_ref[...],
                                            (((1,), (0,)), ((), ())),
                                            precision=_HI) + blj_ref[...]

        row = jnp.sum(g, axis=1, keepdims=True)
        col_row = jax.lax.dot_general(jnp.ones((1, J), jnp.float32), g,
                                      (((1,), (0,)), ((), ())),
                                      precision=_HI)
        rowT_row = jax.lax.dot_general(jnp.ones((1, JPM), jnp.float32), g,
                                       (((1,), (1,)), ((), ())),
                                       precision=_HI)
        left = (jnp.ones((J, J), jnp.float32) - row - rowT_row
                - col_row[:, :J] - aggT(a, onesc))
        leftb = jnp.where(left == 1.0, 1.0, 0.0)
        leftb = jnp.where(jj > ii, leftb, 0.0)
        penl_ref[...] = (1.0 - leftb) * 100000.0
        penr_ref[...] = jnp.broadcast_to(row, (J, M)) * 100000.0

    mf = mf_ref[...]
    part = (jax.lax.dot_general(mf[:, :64], wlma_ref[...],
                                (((1,), (0,)), ((), ())), precision=_DEF)
            + jax.lax.dot_general(mf[:, 64:], wlmb_ref[...],
                                  (((1,), (0,)), ((), ())), precision=_DEF))
    vmach_ref[0] = part + blm_ref[0]


def _finish_kernel(vm_ref, base_ref, penl_ref, penr_ref, val_ref, poss_ref):
    v = vm_ref[...] + base_ref[...]
    val_ref[...] = v
    tl = v[:, :J] - penl_ref[...]
    tr = v[:, J:] - penr_ref[...]
    m = jnp.maximum(jnp.max(tl), jnp.max(tr))
    el = jnp.exp(tl - m)
    er = jnp.exp(tr - m)
    s = jnp.sum(el) + jnp.sum(er)
    poss_ref[:, :J] = el / s
    poss_ref[:, J:] = er / s


@functools.partial(jax.jit, static_argnames=())
def kernel(Graph, h, L, W, P, N, jj_src, jj_dst, jm_src, jm_dst,
           W_jj, b_jj, W_jm, b_jm, W_pool, b_pool, W_self, W_neigh, b_sage,
           W_lj, b_lj, W_lm, b_lm):
    del jj_src, jj_dst, jm_src, jm_dst
    f32 = jnp.float32
    hr = h.reshape(1, J).astype(f32)
    hc = h.reshape(J, 1).astype(f32)
    lr = L.reshape(1, J).astype(f32)
    wpn = jnp.stack([W, P, N]).reshape(1, 3).astype(f32)
    blm_flat = b_lm.reshape(NBLK, 1, BLKW)

    const = lambda shape: pl.BlockSpec(shape, lambda j: tuple(0 for _ in shape))
    term, base, penl, penr, vmach = pl.pallas_call(
        _stream_kernel,
        grid=(NBLK,),
        in_specs=[
            const((J, JPM)), const((1, J)), const((J, 1)), const((1, J)),
            const((1, 3)),
            const((5, 256)), const((1, 256)), const((5, 64)), const((1, 64)),
            const((5, 5)), const((1, 5)), const((5, 1)), const((5, 1)),
            const((1, 1)),
            const((256, JPM)), const((1, JPM)),
            pl.BlockSpec((64, BLKW), lambda j: (0, j)),
            pl.BlockSpec((64, BLKW), lambda j: (1, j)),
            pl.BlockSpec((1, 1, BLKW), lambda j: (j, 0, 0)),
        ],
        out_specs=(
            const((1, 1)), const((J, JPM)), const((J, J)), const((J, M)),
            pl.BlockSpec((1, 1, BLKW), lambda j: (j, 0, 0)),
        ),
        out_shape=(
            jax.ShapeDtypeStruct((1, 1), f32),
            jax.ShapeDtypeStruct((J, JPM), f32),
            jax.ShapeDtypeStruct((J, J), f32),
            jax.ShapeDtypeStruct((J, M), f32),
            jax.ShapeDtypeStruct((NBLK, 1, BLKW), f32),
        ),
        scratch_shapes=[pltpu.VMEM((1, 128), f32)],
    )(Graph, hr, hc, lr, wpn,
      W_jj, b_jj.reshape(1, -1), W_jm, b_jm.reshape(1, -1),
      W_pool, b_pool.reshape(1, -1), W_self, W_neigh, b_sage.reshape(1, 1),
      W_lj, b_lj.reshape(1, -1), W_lm, W_lm, blm_flat)

    vmach2d = vmach.reshape(J, JPM)
    value, poss = pl.pallas_call(
        _finish_kernel,
        out_shape=(jax.ShapeDtypeStruct((J, JPM), f32),
                   jax.ShapeDtypeStruct((J, JPM), f32)),
    )(vmach2d, base, penl, penr)

    return (term, value, poss)
